# Initial kernel scaffold; baseline (speedup 1.0000x reference)
import jax
import jax.numpy as jnp
from jax.experimental import pallas as pl


def kernel(x, neigh_orders, pool_neigh_orders, W1, b1, g1, be1, W2, b2, g2, be2):
    raise NotImplementedError("write your pallas kernel here")


# speedup vs baseline: 1.3991x; 1.3991x over previous
import functools

import jax
import jax.numpy as jnp
from jax import lax
from jax.experimental import pallas as pl
from jax.experimental.pallas import tpu as pltpu
from jax.experimental.pallas import tpu_sc as plsc

IN_CH = 32
OUT_CH = 32
LANES = 128
EPS = 1e-5
NEG_SLOPE = 0.2
_CHUNK = 512
_ROWS = 8192


def _make_sc_gather(m_pad):
    info = plsc.get_sparse_core_info()
    nc, ns = info.num_cores, info.num_subcores
    nw = nc * ns
    b_per_w = m_pad // nw
    assert b_per_w % _CHUNK == 0
    n_steps = b_per_w // _CHUNK
    mesh = plsc.VectorSubcoreMesh(core_axis_name="c", subcore_axis_name="s")

    @functools.partial(
        pl.kernel,
        mesh=mesh,
        out_type=jax.ShapeDtypeStruct((m_pad, LANES), jnp.float32),
        scratch_types=[
            pltpu.VMEM((_CHUNK,), jnp.int32),
            pltpu.VMEM((_CHUNK, LANES), jnp.float32),
            pltpu.SemaphoreType.DMA,
        ],
    )
    def gather_kernel(table_hbm, idx_hbm, out_hbm, idx_v, rows_v, sem):
        wid = lax.axis_index("s") * nc + lax.axis_index("c")
        base = wid * b_per_w
        for i in range(n_steps):
            off = base + i * _CHUNK
            pltpu.sync_copy(idx_hbm.at[pl.ds(off, _CHUNK)], idx_v)
            pltpu.async_copy(table_hbm.at[idx_v], rows_v, sem).wait()
            pltpu.sync_copy(rows_v, out_hbm.at[pl.ds(off, _CHUNK)])

    return gather_kernel


def _pool_mean_tc(p_ref, o_ref):
    c = o_ref.shape[1]
    row = lax.broadcasted_iota(jnp.int32, (7 * c, c), 0)
    col = lax.broadcasted_iota(jnp.int32, (7 * c, c), 1)
    avg = jnp.where((row % c) == col, 1.0 / 7.0, 0.0).astype(jnp.float32)
    o_ref[...] = jnp.dot(p_ref[...], avg, preferred_element_type=jnp.float32)


def _make_conv_stats(v_out):
    def conv_stats(g_ref, w_ref, b_ref, y_ref, s_ref, acc_ref):
        i = pl.program_id(0)
        y = jnp.dot(g_ref[...], w_ref[...],
                    preferred_element_type=jnp.float32) + b_ref[...]
        y_ref[...] = y
        rows = i * _ROWS + lax.broadcasted_iota(jnp.int32, y.shape, 0)
        ym = jnp.where(rows < v_out, y, 0.0)

        @pl.when(i == 0)
        def _():
            acc_ref[...] = jnp.zeros_like(acc_ref)

        acc_ref[0:1, :] += jnp.sum(ym, axis=0, keepdims=True)
        acc_ref[1:2, :] += jnp.sum(ym * ym, axis=0, keepdims=True)
        s_ref[...] = acc_ref[...]

    return conv_stats


def _make_bn_lrelu(v_out):
    def bn_lrelu(y_ref, s_ref, ga_ref, be_ref, o_ref):
        mean = s_ref[0:1, :] / v_out
        var = s_ref[1:2, :] / v_out - mean * mean
        h = ga_ref[...] * (y_ref[...] - mean) * lax.rsqrt(var + EPS)
        h = h + be_ref[...]
        o_ref[...] = jnp.where(h >= 0, h, NEG_SLOPE * h)

    return bn_lrelu


def _pad_lanes(t):
    return jnp.pad(t, ((0, 0), (0, LANES - t.shape[1])))


def kernel(x, neigh_orders, pool_neigh_orders, W1, b1, g1, be1, W2, b2, g2, be2):
    B, c_in, v_in = x.shape
    v_out = neigh_orders.shape[0] // 7
    m = 7 * v_out
    m_pad = -(-m // (32 * _CHUNK)) * (32 * _CHUNK)
    pad = m_pad - m
    pool_idx = jnp.concatenate(
        [pool_neigh_orders, jnp.zeros((pad,), jnp.int32)])
    neigh_idx = jnp.concatenate(
        [neigh_orders, jnp.zeros((pad,), jnp.int32)])

    gather = _make_sc_gather(m_pad)

    nb = -(-v_out // _ROWS)
    full = lambda i: (0, 0)
    blocked = lambda i: (i, 0)
    pool_call = pl.pallas_call(
        _pool_mean_tc,
        grid=(nb,),
        in_specs=[pl.BlockSpec((_ROWS, 7 * c_in), blocked)],
        out_specs=pl.BlockSpec((_ROWS, c_in), blocked),
        out_shape=jax.ShapeDtypeStruct((v_out, c_in), jnp.float32),
    )
    conv_call = pl.pallas_call(
        _make_conv_stats(v_out),
        grid=(nb,),
        in_specs=[
            pl.BlockSpec((_ROWS, 7 * c_in), blocked),
            pl.BlockSpec((7 * c_in, OUT_CH), full),
            pl.BlockSpec((1, OUT_CH), full),
        ],
        out_specs=[
            pl.BlockSpec((_ROWS, OUT_CH), blocked),
            pl.BlockSpec((2, OUT_CH), full),
        ],
        out_shape=[
            jax.ShapeDtypeStruct((v_out, OUT_CH), jnp.float32),
            jax.ShapeDtypeStruct((2, OUT_CH), jnp.float32),
        ],
        scratch_shapes=[pltpu.VMEM((2, OUT_CH), jnp.float32)],
    )
    bn_call = pl.pallas_call(
        _make_bn_lrelu(v_out),
        grid=(nb,),
        in_specs=[
            pl.BlockSpec((_ROWS, OUT_CH), blocked),
            pl.BlockSpec((2, OUT_CH), full),
            pl.BlockSpec((1, OUT_CH), full),
            pl.BlockSpec((1, OUT_CH), full),
        ],
        out_specs=pl.BlockSpec((_ROWS, OUT_CH), blocked),
        out_shape=jax.ShapeDtypeStruct((v_out, OUT_CH), jnp.float32),
    )

    def conv_block(gth, W, b, ga, be):
        y, s = conv_call(gth, W, b.reshape(1, -1))
        return bn_call(y, s, ga.reshape(1, -1), be.reshape(1, -1))

    def one_batch(xb):
        table0 = _pad_lanes(xb.T)
        p = gather(table0, pool_idx)[:m, :c_in].reshape(v_out, 7 * c_in)
        h0 = pool_call(p)
        gth1 = gather(_pad_lanes(h0), neigh_idx)[:m, :c_in]
        h1 = conv_block(gth1.reshape(v_out, 7 * c_in), W1, b1, g1, be1)
        gth2 = gather(_pad_lanes(h1), neigh_idx)[:m, :OUT_CH]
        h2 = conv_block(gth2.reshape(v_out, 7 * OUT_CH), W2, b2, g2, be2)
        return h2.T

    return jnp.stack([one_batch(x[b]) for b in range(B)], axis=0)
